# baseline (speedup 1.0000x reference)
import jax
import jax.numpy as jnp
from jax import lax
from jax.experimental import pallas as pl
from jax.experimental.pallas import tpu as pltpu
from jax.experimental.pallas import tpu_sc as plsc

HIDDEN = 64
LANES = 16
WINDOW = 128
NGBUF = 4
GPB = 4
NOBUF = 2
SUPER = WINDOW * GPB
NWORKERS = 32


def _gather_scale(table, idx_flat):
    n = idx_flat.shape[0]
    per_w = n // NWORKERS
    nchunk = per_w // WINDOW
    nsuper = per_w // SUPER
    mesh = plsc.VectorSubcoreMesh(core_axis_name="c", subcore_axis_name="s")

    @pl.kernel(
        out_type=jax.ShapeDtypeStruct((n, HIDDEN), jnp.float32),
        mesh=mesh,
        compiler_params=pltpu.CompilerParams(use_tc_tiling_on_sc=False),
        scratch_types=(
            [pltpu.VMEM((per_w,), jnp.int32)]
            + [pltpu.VMEM((WINDOW, HIDDEN), jnp.float32)] * NGBUF
            + [pltpu.VMEM((SUPER, HIDDEN), jnp.float32)] * NOBUF
            + [pltpu.SemaphoreType.DMA] * (NGBUF + NOBUF)
        ),
    )
    def k(table_hbm, idx_hbm, out_hbm, idx_v, *rest):
        gbuf = rest[0:NGBUF]
        obuf = rest[NGBUF : NGBUF + NOBUF]
        gsem = rest[NGBUF + NOBUF : 2 * NGBUF + NOBUF]
        wsem = rest[2 * NGBUF + NOBUF : 2 * NGBUF + 2 * NOBUF]

        wid = lax.axis_index("s") * 2 + lax.axis_index("c")
        base = wid * per_w

        pltpu.sync_copy(idx_hbm.at[pl.ds(base, per_w)], idx_v)

        def gather_start(b, g):
            pltpu.make_async_copy(
                table_hbm.at[idx_v.at[pl.ds(g * WINDOW, WINDOW)]],
                gbuf[b],
                gsem[b],
            ).start()

        def gather_wait(b):
            pltpu.make_async_copy(
                table_hbm.at[idx_v.at[pl.ds(0, WINDOW)]], gbuf[b], gsem[b]
            ).wait()

        def write_start(o, s):
            pltpu.make_async_copy(
                obuf[o], out_hbm.at[pl.ds(base + s * SUPER, SUPER)], wsem[o]
            ).start()

        def write_wait(o):
            pltpu.make_async_copy(
                obuf[o], out_hbm.at[pl.ds(base, SUPER)], wsem[o]
            ).wait()

        for b in range(NGBUF):
            gather_start(b, b)

        @pl.loop(0, nsuper, step=NOBUF)
        def _(s0):
            for oo in range(NOBUF):
                s = s0 + oo

                @pl.when(s >= NOBUF)
                def _(oo=oo):
                    write_wait(oo)

                for j in range(GPB):
                    g = s * GPB + j
                    gather_wait(j)
                    gb, ob = gbuf[j], obuf[oo]

                    @pl.loop(0, WINDOW, step=4)
                    def _(r, gb=gb, ob=ob, j=j):
                        for rr in range(4):
                            for c in range(0, HIDDEN, LANES):
                                ob[j * WINDOW + r + rr, pl.ds(c, LANES)] = (
                                    gb[r + rr, pl.ds(c, LANES)] * 8.0
                                )

                    @pl.when(g + NGBUF < nchunk)
                    def _(j=j, g=g):
                        gather_start(j, g + NGBUF)

                write_start(oo, s)

        for o in range(NOBUF):
            write_wait(o)

    return k(table, idx_flat)


@jax.jit
def kernel(x, emb_weight, pe):
    del pe
    b, s = x.shape
    flat = _gather_scale(emb_weight, x.reshape(b * s).astype(jnp.int32))
    return flat.reshape(b, s, HIDDEN)
